# baseline (speedup 1.0000x reference)
import functools

import jax
import jax.numpy as jnp
from jax.experimental import pallas as pl


def _body(x_ref, perm_ref, signs_ref, o_ref):
    rows = x_ref.shape[0]
    idx = jnp.broadcast_to(perm_ref[...][None, :], (rows, perm_ref.shape[0]))
    g = jnp.take_along_axis(x_ref[...], idx, axis=1)
    o_ref[...] = g * signs_ref[...][None, :]


def kernel(x, perm, signs):
    b, s, d = x.shape
    rows = b * s
    xf = x.reshape(rows, d)
    rb = 1024
    grid = (rows // rb,)
    out = pl.pallas_call(
        _body,
        grid=grid,
        in_specs=[
            pl.BlockSpec((rb, d), lambda i: (i, 0)),
            pl.BlockSpec((d,), lambda i: (0,)),
            pl.BlockSpec((d,), lambda i: (0,)),
        ],
        out_specs=pl.BlockSpec((rb, d), lambda i: (i, 0)),
        out_shape=jax.ShapeDtypeStruct((rows, d), jnp.float32),
    )(xf, perm, signs)
    return out.reshape(b, s, d)


# speedup vs baseline: 2.0497x; 2.0497x over previous
import jax
import jax.numpy as jnp
from jax.experimental import pallas as pl


def _body(x_ref, perm_ref, signs_ref, o_ref):
    rb, s, d = x_ref.shape
    idx = jnp.broadcast_to(perm_ref[...][None, None, :], (rb, s, d))
    g = jnp.take_along_axis(x_ref[...], idx, axis=2)
    o_ref[...] = g * signs_ref[...][None, None, :]


def kernel(x, perm, signs):
    b, s, d = x.shape
    rb = 32
    grid = (b // rb,)
    return pl.pallas_call(
        _body,
        grid=grid,
        in_specs=[
            pl.BlockSpec((rb, s, d), lambda i: (i, 0, 0)),
            pl.BlockSpec((d,), lambda i: (0,)),
            pl.BlockSpec((d,), lambda i: (0,)),
        ],
        out_specs=pl.BlockSpec((rb, s, d), lambda i: (i, 0, 0)),
        out_shape=jax.ShapeDtypeStruct((b, s, d), jnp.float32),
    )(x, perm, signs)


# speedup vs baseline: 2.3787x; 1.1605x over previous
import jax
import jax.numpy as jnp
from jax.experimental import pallas as pl


def _body(x_ref, perm_ref, signs_ref, o_ref):
    rb, s, d = x_ref.shape
    idx = jnp.broadcast_to(perm_ref[...][None, None, :], (rb, s, d))
    g = jnp.take_along_axis(x_ref[...], idx, axis=2)
    o_ref[...] = g * signs_ref[...][None, None, :]


def kernel(x, perm, signs):
    b, s, d = x.shape
    rb = 64
    grid = (b // rb,)
    return pl.pallas_call(
        _body,
        grid=grid,
        in_specs=[
            pl.BlockSpec((rb, s, d), lambda i: (i, 0, 0)),
            pl.BlockSpec((d,), lambda i: (0,)),
            pl.BlockSpec((d,), lambda i: (0,)),
        ],
        out_specs=pl.BlockSpec((rb, s, d), lambda i: (i, 0, 0)),
        out_shape=jax.ShapeDtypeStruct((b, s, d), jnp.float32),
    )(x, perm, signs)


# speedup vs baseline: 2.5568x; 1.0749x over previous
import jax
import jax.numpy as jnp
from jax.experimental import pallas as pl


def _body(x_ref, perm_ref, signs_ref, o_ref):
    rb, s, d = x_ref.shape
    idx = jnp.broadcast_to(perm_ref[...][None, None, :], (rb, s, d))
    g = jnp.take_along_axis(x_ref[...], idx, axis=2)
    o_ref[...] = g * signs_ref[...][None, None, :]


def kernel(x, perm, signs):
    b, s, d = x.shape
    rb = 128
    grid = (b // rb,)
    return pl.pallas_call(
        _body,
        grid=grid,
        in_specs=[
            pl.BlockSpec((rb, s, d), lambda i: (i, 0, 0)),
            pl.BlockSpec((d,), lambda i: (0,)),
            pl.BlockSpec((d,), lambda i: (0,)),
        ],
        out_specs=pl.BlockSpec((rb, s, d), lambda i: (i, 0, 0)),
        out_shape=jax.ShapeDtypeStruct((b, s, d), jnp.float32),
    )(x, perm, signs)


# speedup vs baseline: 2.5809x; 1.0094x over previous
import jax
import jax.numpy as jnp
from jax.experimental import pallas as pl


def _body(x_ref, perm_ref, signs_ref, o_ref):
    rb, s, d = x_ref.shape
    idx = jnp.broadcast_to(perm_ref[...][None, None, :], (rb, s, d))
    g = jnp.take_along_axis(x_ref[...], idx, axis=2)
    o_ref[...] = g * signs_ref[...][None, None, :]


def kernel(x, perm, signs):
    b, s, d = x.shape
    rb = 256
    grid = (b // rb,)
    return pl.pallas_call(
        _body,
        grid=grid,
        in_specs=[
            pl.BlockSpec((rb, s, d), lambda i: (i, 0, 0)),
            pl.BlockSpec((d,), lambda i: (0,)),
            pl.BlockSpec((d,), lambda i: (0,)),
        ],
        out_specs=pl.BlockSpec((rb, s, d), lambda i: (i, 0, 0)),
        out_shape=jax.ShapeDtypeStruct((b, s, d), jnp.float32),
    )(x, perm, signs)


# speedup vs baseline: 2.5955x; 1.0057x over previous
import jax
import jax.numpy as jnp
from jax.experimental import pallas as pl


def _body(x_ref, perm_ref, signs_ref, o_ref):
    rb, s, d = x_ref.shape
    idx = jnp.broadcast_to(perm_ref[...][None, None, :], (rb, s, d))
    g = jnp.take_along_axis(x_ref[...], idx, axis=2)
    o_ref[...] = g * signs_ref[...][None, None, :]


def kernel(x, perm, signs):
    b, s, d = x.shape
    rb = 512
    grid = (b // rb,)
    return pl.pallas_call(
        _body,
        grid=grid,
        in_specs=[
            pl.BlockSpec((rb, s, d), lambda i: (i, 0, 0)),
            pl.BlockSpec((d,), lambda i: (0,)),
            pl.BlockSpec((d,), lambda i: (0,)),
        ],
        out_specs=pl.BlockSpec((rb, s, d), lambda i: (i, 0, 0)),
        out_shape=jax.ShapeDtypeStruct((b, s, d), jnp.float32),
    )(x, perm, signs)
